# baseline (speedup 1.0000x reference)
import jax
import jax.numpy as jnp
from jax import lax
from jax.experimental import pallas as pl
from jax.experimental.pallas import tpu as pltpu
from jax.experimental.pallas import tpu_sc as plsc

_NC, _NS = 2, 16
_NW = _NC * _NS
_SPAD = 56
_C = 224
_NBUF = 4


def kernel(x, weight):
    B, S = x.shape
    V, D = weight.shape
    n = B * _SPAD
    per_tile = n // _NW
    nchunks = per_tile // _C
    ngroups = nchunks // _NBUF
    xi = x.astype(jnp.int32)
    idx = jnp.pad(xi, ((0, 0), (0, _SPAD - S))).reshape(n)

    mesh = plsc.VectorSubcoreMesh(core_axis_name="c", subcore_axis_name="s")

    @pl.kernel(
        out_type=jax.ShapeDtypeStruct((n, D), weight.dtype),
        mesh=mesh,
        scratch_types=[
            pltpu.VMEM((per_tile,), jnp.int32),
            pltpu.VMEM((_NBUF, _C, D), jnp.float32),
            pltpu.SemaphoreType.DMA,
            pltpu.SemaphoreType.DMA,
            pltpu.SemaphoreType.DMA,
            pltpu.SemaphoreType.DMA,
            pltpu.SemaphoreType.DMA,
            pltpu.SemaphoreType.DMA,
            pltpu.SemaphoreType.DMA,
            pltpu.SemaphoreType.DMA,
        ],
    )
    def k(w_hbm, i_hbm, o_hbm, idx_v, bufs, g0, g1, g2, g3, w0, w1, w2, w3):
        gsems = (g0, g1, g2, g3)
        wsems = (w0, w1, w2, w3)
        wid = lax.axis_index("s") * _NC + lax.axis_index("c")
        base = wid * per_tile
        pltpu.sync_copy(i_hbm.at[pl.ds(base, per_tile)], idx_v)

        def gather_copy(c, b):
            return pltpu.make_async_copy(
                w_hbm.at[idx_v.at[pl.ds(c * _C, _C)]], bufs.at[b], gsems[b])

        def write_copy(c, b):
            return pltpu.make_async_copy(
                bufs.at[b], o_hbm.at[pl.ds(base + c * _C, _C)], wsems[b])

        for b in range(_NBUF):
            gather_copy(b, b).start()

        @pl.loop(0, ngroups - 1)
        def _(g):
            for b in range(_NBUF):
                c = g * _NBUF + b
                gather_copy(c, b).wait()
                write_copy(c, b).start()
            for b in range(_NBUF):
                c = g * _NBUF + b
                write_copy(c, b).wait()
                gather_copy(c + _NBUF, b).start()

        gl = ngroups - 1
        for b in range(_NBUF):
            c = gl * _NBUF + b
            gather_copy(c, b).wait()
            write_copy(c, b).start()
        for b in range(_NBUF):
            write_copy(gl * _NBUF + b, b).wait()

    out = k(weight, idx).reshape(B, _SPAD, D)
    return out[:, :S, :]


# speedup vs baseline: 6.2904x; 6.2904x over previous
import jax
import jax.numpy as jnp
from jax import lax
from jax.experimental import pallas as pl
from jax.experimental.pallas import tpu as pltpu
from jax.experimental.pallas import tpu_sc as plsc

_NC, _NS = 2, 16
_NW = _NC * _NS
_SPAD = 56
_C = 224
_NBUF = 4


def kernel(x, weight):
    B, S = x.shape
    V, D = weight.shape
    n = B * _SPAD
    per_tile = n // _NW
    nchunks = per_tile // _C
    ngroups = nchunks // _NBUF
    xi = x.astype(jnp.int32)
    idx = jnp.concatenate([xi, xi[:, : _SPAD - S]], axis=1).reshape(n)

    mesh = plsc.VectorSubcoreMesh(core_axis_name="c", subcore_axis_name="s")

    @pl.kernel(
        out_type=jax.ShapeDtypeStruct((n, D), weight.dtype),
        mesh=mesh,
        scratch_types=[
            pltpu.VMEM((per_tile,), jnp.int32),
            pltpu.VMEM((_NBUF, _C, D), jnp.float32),
            pltpu.SemaphoreType.DMA,
            pltpu.SemaphoreType.DMA,
            pltpu.SemaphoreType.DMA,
            pltpu.SemaphoreType.DMA,
            pltpu.SemaphoreType.DMA,
            pltpu.SemaphoreType.DMA,
            pltpu.SemaphoreType.DMA,
            pltpu.SemaphoreType.DMA,
        ],
    )
    def k(w_hbm, i_hbm, o_hbm, idx_v, bufs, g0, g1, g2, g3, w0, w1, w2, w3):
        gsems = (g0, g1, g2, g3)
        wsems = (w0, w1, w2, w3)
        wid = lax.axis_index("s") * _NC + lax.axis_index("c")
        base = wid * per_tile
        pltpu.sync_copy(i_hbm.at[pl.ds(base, per_tile)], idx_v)

        def gather_copy(c, b):
            return pltpu.make_async_copy(
                w_hbm.at[idx_v.at[pl.ds(c * _C, _C)]], bufs.at[b], gsems[b])

        def write_copy(c, b):
            return pltpu.make_async_copy(
                bufs.at[b], o_hbm.at[pl.ds(base + c * _C, _C)], wsems[b])

        for b in range(_NBUF):
            gather_copy(b, b).start()

        @pl.loop(0, ngroups - 1)
        def _(g):
            for b in range(_NBUF):
                c = g * _NBUF + b
                gather_copy(c, b).wait()
                write_copy(c, b).start()
            for b in range(_NBUF):
                c = g * _NBUF + b
                write_copy(c, b).wait()
                gather_copy(c + _NBUF, b).start()

        gl = ngroups - 1
        for b in range(_NBUF):
            c = gl * _NBUF + b
            gather_copy(c, b).wait()
            write_copy(c, b).start()
        for b in range(_NBUF):
            write_copy(gl * _NBUF + b, b).wait()

    out = k(weight, idx).reshape(B, _SPAD, D)
    return out[:, :S, :]
